# baseline (speedup 1.0000x reference)
import functools

import jax
import jax.numpy as jnp
from jax.experimental import pallas as pl
from jax.experimental.pallas import tpu as pltpu

N, D, E = 8192, 4096, 64
ROWS = 512
NBUF = 4


def _router_body(x_hbm, w_ref, b_ref,
                 probs_ref, idx_ref, tprob_ref, aux_ref, me_ref, ce_ref,
                 ent_ref, xbuf, sems):
    i = pl.program_id(0)
    nsteps = pl.num_programs(0)

    def x_copy(block, slot):
        return pltpu.make_async_copy(
            x_hbm.at[pl.ds(block * ROWS, ROWS), :], xbuf.at[slot],
            sems.at[slot])

    @pl.when(i == 0)
    def _prime():
        for j in range(NBUF):
            x_copy(j, j).start()

    s = jax.lax.rem(i, NBUF)
    x_copy(i, s).wait()

    logits = jnp.dot(xbuf[s], w_ref[...],
                     preferred_element_type=jnp.float32) + b_ref[...]

    @pl.when(i + NBUF < nsteps)
    def _next():
        x_copy(i + NBUF, s).start()

    m = jnp.max(logits, axis=-1, keepdims=True)
    d = logits - m
    ex = jnp.exp(d)
    s_full = jnp.dot(ex, jnp.ones((E, E), jnp.float32),
                     preferred_element_type=jnp.float32)
    rinv = 1.0 / s_full
    p = ex * rinv
    probs_ref[...] = p

    lane_f = jax.lax.broadcasted_iota(
        jnp.int32, logits.shape, 1).astype(jnp.float32)
    idx_col = jnp.min(jnp.where(d >= 0.0, lane_f, jnp.float32(E)),
                      axis=-1, keepdims=True)
    idx_ref[...] = idx_col.astype(jnp.int32)
    tprob_ref[...] = rinv[:, :1]

    one_hot = (d >= 0.0).astype(jnp.float32)
    me_part = jnp.sum(one_hot, axis=0, keepdims=True) * (1.0 / N)
    ce_part = jnp.sum(p, axis=0, keepdims=True) * (1.0 / N)
    ent_col = jnp.log(s_full[:, :1]) - jnp.sum(p * d, axis=-1,
                                               keepdims=True)
    ent_part = (jnp.sum(ent_col) * (1.0 / N)).reshape(1, 1)

    @pl.when(i == 0)
    def _init():
        me_ref[...] = me_part
        ce_ref[...] = ce_part
        ent_ref[...] = ent_part

    @pl.when(i > 0)
    def _acc():
        me_ref[...] += me_part
        ce_ref[...] += ce_part
        ent_ref[...] += ent_part

    @pl.when(i == nsteps - 1)
    def _finish():
        aux_ref[...] = 0.05 * E * jnp.sum(
            me_ref[...] * ce_ref[...]).reshape(1, 1)


@functools.partial(jax.jit, static_argnames=())
def kernel(x, W, b):
    nsteps = N // ROWS
    b2 = b.reshape(1, E)
    out_types = (
        jax.ShapeDtypeStruct((N, E), jnp.float32),
        jax.ShapeDtypeStruct((N, 1), jnp.int32),
        jax.ShapeDtypeStruct((N, 1), jnp.float32),
        jax.ShapeDtypeStruct((1, 1), jnp.float32),
        jax.ShapeDtypeStruct((1, E), jnp.float32),
        jax.ShapeDtypeStruct((1, E), jnp.float32),
        jax.ShapeDtypeStruct((1, 1), jnp.float32),
    )
    probs, idx2, tp2, aux, me, ce, ent = pl.pallas_call(
        _router_body,
        grid=(nsteps,),
        in_specs=[
            pl.BlockSpec(memory_space=pl.ANY),
            pl.BlockSpec((D, E), lambda i: (0, 0)),
            pl.BlockSpec((1, E), lambda i: (0, 0)),
        ],
        out_specs=[
            pl.BlockSpec((ROWS, E), lambda i: (i, 0)),
            pl.BlockSpec((ROWS, 1), lambda i: (i, 0)),
            pl.BlockSpec((ROWS, 1), lambda i: (i, 0)),
            pl.BlockSpec((1, 1), lambda i: (0, 0)),
            pl.BlockSpec((1, E), lambda i: (0, 0)),
            pl.BlockSpec((1, E), lambda i: (0, 0)),
            pl.BlockSpec((1, 1), lambda i: (0, 0)),
        ],
        scratch_shapes=[
            pltpu.VMEM((NBUF, ROWS, D), jnp.float32),
            pltpu.SemaphoreType.DMA((NBUF,)),
        ],
        out_shape=out_types)(x, W, b2)
    return (probs, idx2.reshape(N), tp2.reshape(N), aux[0, 0],
            me[0], ce[0], ent[0, 0])


# speedup vs baseline: 1.0082x; 1.0082x over previous
import functools

import jax
import jax.numpy as jnp
from jax.experimental import pallas as pl

N, D, E = 8192, 4096, 64
ROWS = 1024


def _router_body(x_ref, w_ref, b_ref,
                 probs_ref, idx_ref, tprob_ref, aux_ref, me_ref, ce_ref,
                 ent_ref):
    i = pl.program_id(0)
    nsteps = pl.num_programs(0)

    logits = jnp.dot(x_ref[...], w_ref[...],
                     preferred_element_type=jnp.float32) + b_ref[...]
    m = jnp.max(logits, axis=-1, keepdims=True)
    d = logits - m
    ex = jnp.exp(d)
    s_full = jnp.dot(ex, jnp.ones((E, E), jnp.float32),
                     preferred_element_type=jnp.float32)
    rinv = 1.0 / s_full
    p = ex * rinv
    probs_ref[...] = p

    lane_f = jax.lax.broadcasted_iota(
        jnp.int32, logits.shape, 1).astype(jnp.float32)
    idx_col = jnp.min(jnp.where(d >= 0.0, lane_f, jnp.float32(E)),
                      axis=-1, keepdims=True)
    idx_ref[...] = idx_col.astype(jnp.int32)
    tprob_ref[...] = rinv[:, :1]

    one_hot = (d >= 0.0).astype(jnp.float32)
    me_part = jnp.sum(one_hot, axis=0, keepdims=True) * (1.0 / N)
    ce_part = jnp.sum(p, axis=0, keepdims=True) * (1.0 / N)
    ent_col = jnp.log(s_full[:, :1]) - jnp.sum(p * d, axis=-1,
                                               keepdims=True)
    ent_part = (jnp.sum(ent_col) * (1.0 / N)).reshape(1, 1)

    @pl.when(i == 0)
    def _init():
        me_ref[...] = me_part
        ce_ref[...] = ce_part
        ent_ref[...] = ent_part

    @pl.when(i > 0)
    def _acc():
        me_ref[...] += me_part
        ce_ref[...] += ce_part
        ent_ref[...] += ent_part

    @pl.when(i == nsteps - 1)
    def _finish():
        aux_ref[...] = 0.05 * E * jnp.sum(
            me_ref[...] * ce_ref[...]).reshape(1, 1)


@functools.partial(jax.jit, static_argnames=())
def kernel(x, W, b):
    nsteps = N // ROWS
    b2 = b.reshape(1, E)
    out_types = (
        jax.ShapeDtypeStruct((N, E), jnp.float32),
        jax.ShapeDtypeStruct((N, 1), jnp.int32),
        jax.ShapeDtypeStruct((N, 1), jnp.float32),
        jax.ShapeDtypeStruct((1, 1), jnp.float32),
        jax.ShapeDtypeStruct((1, E), jnp.float32),
        jax.ShapeDtypeStruct((1, E), jnp.float32),
        jax.ShapeDtypeStruct((1, 1), jnp.float32),
    )
    grid_spec = pl.GridSpec(
        grid=(nsteps,),
        in_specs=[
            pl.BlockSpec((ROWS, D), lambda i: (i, 0)),
            pl.BlockSpec((D, E), lambda i: (0, 0)),
            pl.BlockSpec((1, E), lambda i: (0, 0)),
        ],
        out_specs=[
            pl.BlockSpec((ROWS, E), lambda i: (i, 0)),
            pl.BlockSpec((ROWS, 1), lambda i: (i, 0)),
            pl.BlockSpec((ROWS, 1), lambda i: (i, 0)),
            pl.BlockSpec((1, 1), lambda i: (0, 0)),
            pl.BlockSpec((1, E), lambda i: (0, 0)),
            pl.BlockSpec((1, E), lambda i: (0, 0)),
            pl.BlockSpec((1, 1), lambda i: (0, 0)),
        ],
    )
    probs, idx2, tp2, aux, me, ce, ent = pl.pallas_call(
        _router_body, grid_spec=grid_spec, out_shape=out_types)(x, W, b2)
    return (probs, idx2.reshape(N), tp2.reshape(N), aux[0, 0],
            me[0], ce[0], ent[0, 0])
